# baseline (speedup 1.0000x reference)
import jax
import jax.numpy as jnp
from jax import lax
from jax.experimental import pallas as pl
from jax.experimental.pallas import tpu as pltpu
from jax.experimental.pallas import tpu_sc as plsc

B = 8
L = 2048
PATCH = 16
T = 128
D_MODEL = 128
MIN_PATCH = 4
THRESHOLD_FACTOR = 1.5
EPS = 1e-5
NBLK = L // 16
ROWS_PER_CORE = 4
WPR = 4
TOK_PER_W = T // WPR
SEG_PER_W = TOK_PER_W * PATCH
SH_STRIDE = 160
BIG = 2 * L


def _gather(ref, idx):
    return plsc.load_gather(ref, [idx])


def _scatter(ref, idx, x, mask=None):
    plsc.store_scatter(ref, [idx], x, mask=mask)


def _axes():
    return lax.axis_index("c"), lax.axis_index("s")


_when = pl.when
_while = lax.while_loop
_fori = lax.fori_loop
_copy = pltpu.sync_copy


def _barrier():
    plsc.subcore_barrier()


def _sc_body(x_hbm, seg_hbm, cnt_hbm, xr, dv, segv, bnd, cntv, shared):
    c, s = _axes()
    row = c * ROWS_PER_CORE + s // WPR
    r4 = s // WPR
    q = s % WPR
    lanes = lax.iota(jnp.int32, 16)

    _copy(x_hbm.at[pl.ds(row * L, L)], xr.at[pl.ds(0, L)])

    @_when(q == 0)
    def _():
        acc = jnp.zeros((16,), jnp.float32)
        for k in range(NBLK):
            a = xr[pl.ds(k * 16, 16)]
            bshift = _gather(xr, lanes + (k * 16 + 1))
            v = jnp.abs(bshift - a)
            ok = (lanes + k * 16) < (L - 1)
            dv[pl.ds(k * 16, 16)] = jnp.where(ok, v, -1.0)
            acc = acc + jnp.where(ok, v, 0.0)
        thr = jnp.sum(acc) * jnp.float32(THRESHOLD_FACTOR / (L - 1))

        for k in range(9):
            bnd[pl.ds(k * 16, 16)] = jnp.zeros((16,), jnp.int32)

        def cond(st):
            k, last, cnt = st
            return (k < NBLK) & (cnt <= T)

        def body(st):
            k, last, cnt = st
            d_blk = _gather(dv, lanes + k * 16)
            pos = lanes + (k * 16 + 1)
            m = (d_blk > thr) & (pos >= last + MIN_PATCH)
            c1 = jnp.min(jnp.where(m, pos, BIG))
            c2 = jnp.min(jnp.where(m & (pos >= c1 + MIN_PATCH), pos, BIG))
            c3 = jnp.min(jnp.where(m & (pos >= c2 + MIN_PATCH), pos, BIG))
            c4 = jnp.min(jnp.where(m & (pos >= c3 + MIN_PATCH), pos, BIG))
            nacc = ((c1 < BIG).astype(jnp.int32) + (c2 < BIG).astype(jnp.int32)
                    + (c3 < BIG).astype(jnp.int32) + (c4 < BIG).astype(jnp.int32))
            vals = jnp.where(lanes == 0, c1,
                             jnp.where(lanes == 1, c2,
                                       jnp.where(lanes == 2, c3, c4)))
            _scatter(bnd, cnt + lanes, vals,
                     mask=(lanes < nacc) & ((cnt + lanes) <= T))
            new_last = jnp.where(c4 < BIG, c4,
                                 jnp.where(c3 < BIG, c3,
                                           jnp.where(c2 < BIG, c2,
                                                     jnp.where(c1 < BIG, c1,
                                                               last))))
            return (k + 1, new_last, cnt + nacc)

        _, _, cnt = _while(
            cond, body, (jnp.int32(0), jnp.int32(0), jnp.int32(1)))

        _scatter(bnd, jnp.full((16,), cnt, jnp.int32),
                 jnp.full((16,), jnp.int32(L), jnp.int32),
                 mask=(lanes == 0) & (cnt <= T))

        cntv[...] = jnp.full((16,), cnt, jnp.int32)
        _copy(bnd, shared.at[pl.ds(r4 * SH_STRIDE, 144)])
        _copy(cntv, shared.at[pl.ds(r4 * SH_STRIDE + 144, 16)])

    _barrier()

    _copy(shared.at[pl.ds(r4 * SH_STRIDE, 144)], bnd)
    _copy(shared.at[pl.ds(r4 * SH_STRIDE + 144, 16)], cntv)
    cntvec = cntv[...]

    tok0 = q * TOK_PER_W
    for bi in range(TOK_PER_W // 16):
        tbase = tok0 + bi * 16
        s_v = _gather(bnd, lanes + tbase)
        e_v = _gather(bnd, lanes + tbase + 1)
        seg_len = jnp.maximum(e_v - s_v, 1)
        sf = seg_len.astype(jnp.float32)
        scale = sf * (1.0 / PATCH)
        hi_cap = seg_len - 1
        hi_cap_f = hi_cap.astype(jnp.float32)
        valid = (lanes + tbase) < cntvec
        for j in range(PATCH):
            src = (j + 0.5) * scale - 0.5
            src = jnp.minimum(jnp.maximum(src, 0.0), hi_cap_f)
            lo = src.astype(jnp.int32)
            hi = jnp.minimum(lo + 1, hi_cap)
            w = src - lo.astype(jnp.float32)
            gl = _gather(xr, s_v + lo)
            gh = _gather(xr, s_v + hi)
            res = gl * (1.0 - w) + gh * w
            res = jnp.where(valid, res, 0.0)
            _scatter(segv, lanes * PATCH + (bi * 256 + j), res)

    _copy(segv, seg_hbm.at[pl.ds(row * (T * PATCH) + q * SEG_PER_W,
                                 SEG_PER_W)])

    @_when(q == 0)
    def _():
        _copy(cntv, cnt_hbm.at[pl.ds(row * 16, 16)])


_SC_STAGE_CACHE = {}


def _sc_stage_fn():
    if "k" not in _SC_STAGE_CACHE:
        _SC_STAGE_CACHE["k"] = pl.kernel(
            _sc_body,
            out_type=(
                jax.ShapeDtypeStruct((B * T * PATCH,), jnp.float32),
                jax.ShapeDtypeStruct((B * 16,), jnp.int32),
            ),
            mesh=plsc.VectorSubcoreMesh(
                core_axis_name="c", subcore_axis_name="s"),
            scratch_types=[
                pltpu.VMEM((L + 16,), jnp.float32),
                pltpu.VMEM((L,), jnp.float32),
                pltpu.VMEM((SEG_PER_W,), jnp.float32),
                pltpu.VMEM((144,), jnp.int32),
                pltpu.VMEM((16,), jnp.int32),
                pltpu.VMEM_SHARED((ROWS_PER_CORE * SH_STRIDE,), jnp.int32),
            ],
            compiler_params=pltpu.CompilerParams(needs_layout_passes=False),
        )
    return _SC_STAGE_CACHE["k"]


def _tc_body(seg_ref, cnt_ref, w_ref, b_ref, g_ref, beta_ref, tnp_ref, o_ref):
    seg = seg_ref[...]
    wmat = w_ref[...]
    pe = lax.dot_general(
        seg, wmat, (((2,), (0,)), ((), ())),
        preferred_element_type=jnp.float32)
    pe = pe + b_ref[...][None, None, :]
    cnt3 = lax.broadcast_in_dim(cnt_ref[:, 0:1], (B, T, D_MODEL), (0, 1))
    tok3 = lax.broadcasted_iota(jnp.int32, (B, T, D_MODEL), 1)
    valid3 = (tok3 < cnt3) & (tok3 < tnp_ref[0])
    pe = jnp.where(valid3, pe, 0.0)
    mean = jnp.mean(pe, axis=-1, keepdims=True)
    cen = pe - mean
    var = jnp.mean(cen * cen, axis=-1, keepdims=True)
    o_ref[...] = cen * lax.rsqrt(var + EPS) * g_ref[...] + beta_ref[...]


def kernel(x, target_n_patches, W, b, gamma, beta):
    seg, cnt = _sc_stage_fn()(x.reshape(B * L))
    seg3 = seg.reshape(B, T, PATCH)
    cnt2 = cnt.reshape(B, 16)
    tnp = jnp.asarray(target_n_patches, jnp.int32).reshape(1)
    out = pl.pallas_call(
        _tc_body,
        out_shape=jax.ShapeDtypeStruct((B, T, D_MODEL), jnp.float32),
        in_specs=[
            pl.BlockSpec(memory_space=pltpu.VMEM),
            pl.BlockSpec(memory_space=pltpu.VMEM),
            pl.BlockSpec(memory_space=pltpu.VMEM),
            pl.BlockSpec(memory_space=pltpu.VMEM),
            pl.BlockSpec(memory_space=pltpu.VMEM),
            pl.BlockSpec(memory_space=pltpu.VMEM),
            pl.BlockSpec(memory_space=pltpu.SMEM),
        ],
    )(seg3, cnt2, W, b, gamma, beta, tnp)
    return out


# speedup vs baseline: 1.0579x; 1.0579x over previous
import jax
import jax.numpy as jnp
from jax import lax
from jax.experimental import pallas as pl
from jax.experimental.pallas import tpu as pltpu
from jax.experimental.pallas import tpu_sc as plsc

B = 8
L = 2048
PATCH = 16
T = 128
D_MODEL = 128
MIN_PATCH = 4
THRESHOLD_FACTOR = 1.5
EPS = 1e-5
NBLK = L // 16
ROWS_PER_CORE = 4
WPR = 4
PROWS = 24
JPW = PROWS // WPR
SEG_PER_W = JPW * T
SH_STRIDE = 160
BIG = 2 * L


def _gather(ref, idx):
    return plsc.load_gather(ref, [idx])


def _scatter(ref, idx, x, mask=None):
    plsc.store_scatter(ref, [idx], x, mask=mask)


def _axes():
    return lax.axis_index("c"), lax.axis_index("s")


_when = pl.when
_while = lax.while_loop
_fori = lax.fori_loop
_copy = pltpu.sync_copy


def _barrier():
    plsc.subcore_barrier()


def _sc_body(x_hbm, seg_hbm, xr, dv, segv, bnd, cntv, shared):
    c, s = _axes()
    r4 = s // WPR
    row = c * ROWS_PER_CORE + r4
    q = s % WPR
    lanes = lax.iota(jnp.int32, 16)

    _copy(x_hbm.at[row], xr.at[pl.ds(0, L)])

    @_when(q == 3)
    def _():
        acc = jnp.zeros((16,), jnp.float32)
        for k in range(NBLK):
            a = xr[pl.ds(k * 16, 16)]
            bshift = _gather(xr, lanes + (k * 16 + 1))
            v = jnp.abs(bshift - a)
            ok = (lanes + k * 16) < (L - 1)
            dv[pl.ds(k * 16, 16)] = jnp.where(ok, v, -1.0)
            acc = acc + jnp.where(ok, v, 0.0)
        thr = jnp.sum(acc) * jnp.float32(THRESHOLD_FACTOR / (L - 1))

        for k in range(9):
            bnd[pl.ds(k * 16, 16)] = jnp.zeros((16,), jnp.int32)

        def cond(st):
            k, last, cnt = st
            return (k < NBLK) & (cnt <= T)

        def body(st):
            k, last, cnt = st
            d_blk = _gather(dv, lanes + k * 16)
            pos = lanes + (k * 16 + 1)
            m = (d_blk > thr) & (pos >= last + MIN_PATCH)
            c1 = jnp.min(jnp.where(m, pos, BIG))
            c2 = jnp.min(jnp.where(m & (pos >= c1 + MIN_PATCH), pos, BIG))
            c3 = jnp.min(jnp.where(m & (pos >= c2 + MIN_PATCH), pos, BIG))
            c4 = jnp.min(jnp.where(m & (pos >= c3 + MIN_PATCH), pos, BIG))
            nacc = ((c1 < BIG).astype(jnp.int32) + (c2 < BIG).astype(jnp.int32)
                    + (c3 < BIG).astype(jnp.int32) + (c4 < BIG).astype(jnp.int32))
            vals = jnp.where(lanes == 0, c1,
                             jnp.where(lanes == 1, c2,
                                       jnp.where(lanes == 2, c3, c4)))
            _scatter(bnd, cnt + lanes, vals,
                     mask=(lanes < nacc) & ((cnt + lanes) <= T))
            new_last = jnp.where(c4 < BIG, c4,
                                 jnp.where(c3 < BIG, c3,
                                           jnp.where(c2 < BIG, c2,
                                                     jnp.where(c1 < BIG, c1,
                                                               last))))
            return (k + 1, new_last, cnt + nacc)

        _, _, cnt = _while(
            cond, body, (jnp.int32(0), jnp.int32(0), jnp.int32(1)))

        _scatter(bnd, jnp.full((16,), cnt, jnp.int32),
                 jnp.full((16,), jnp.int32(L), jnp.int32),
                 mask=(lanes == 0) & (cnt <= T))

        cntv[...] = jnp.full((16,), cnt, jnp.int32)
        _copy(bnd, shared.at[pl.ds(r4 * SH_STRIDE, 144)])
        _copy(cntv, shared.at[pl.ds(r4 * SH_STRIDE + 144, 16)])

    _barrier()

    _copy(shared.at[pl.ds(r4 * SH_STRIDE, 144)], bnd)
    _copy(shared.at[pl.ds(r4 * SH_STRIDE + 144, 16)], cntv)
    cntvec = cntv[...]

    zeros16 = jnp.zeros((16,), jnp.float32)

    @_when(q < 3)
    def _():
        for blk in range(T // 16):
            tbase = blk * 16
            s_v = _gather(bnd, lanes + tbase)
            e_v = _gather(bnd, lanes + tbase + 1)
            seg_len = jnp.maximum(e_v - s_v, 1)
            sf = seg_len.astype(jnp.float32)
            scale = sf * (1.0 / PATCH)
            hi_cap = seg_len - 1
            hi_cap_f = hi_cap.astype(jnp.float32)
            valid = (lanes + tbase) < cntvec
            njreal = jnp.where(q == 2, 4, JPW)
            for jj in range(JPW):
                jf = (q * JPW + jj).astype(jnp.float32)
                src = (jf + 0.5) * scale - 0.5
                src = jnp.minimum(jnp.maximum(src, 0.0), hi_cap_f)
                lo = src.astype(jnp.int32)
                hi = jnp.minimum(lo + 1, hi_cap)
                w = src - lo.astype(jnp.float32)
                gl = _gather(xr, s_v + lo)
                gh = _gather(xr, s_v + hi)
                res = gl * (1.0 - w) + gh * w
                res = jnp.where(valid, res, 0.0)
                res = jnp.where(jj < njreal, res,
                                jnp.where((jj == 4) & valid, 1.0, 0.0))
                segv[pl.ds(jj * T + tbase, 16)] = res

    @_when(q == 3)
    def _():
        for i in range(SEG_PER_W // 16):
            segv[pl.ds(i * 16, 16)] = zeros16

    _copy(segv, seg_hbm.at[pl.ds(row * (PROWS * T) + q * SEG_PER_W,
                                 SEG_PER_W)])


_SC_STAGE_CACHE = {}


def _sc_stage_fn():
    if "k" not in _SC_STAGE_CACHE:
        _SC_STAGE_CACHE["k"] = pl.kernel(
            _sc_body,
            out_type=jax.ShapeDtypeStruct((B * PROWS * T,), jnp.float32),
            mesh=plsc.VectorSubcoreMesh(
                core_axis_name="c", subcore_axis_name="s"),
            scratch_types=[
                pltpu.VMEM((L + 16,), jnp.float32),
                pltpu.VMEM((L,), jnp.float32),
                pltpu.VMEM((SEG_PER_W,), jnp.float32),
                pltpu.VMEM((144,), jnp.int32),
                pltpu.VMEM((16,), jnp.int32),
                pltpu.VMEM_SHARED((ROWS_PER_CORE * SH_STRIDE,), jnp.int32),
            ],
            compiler_params=pltpu.CompilerParams(needs_layout_passes=False),
        )
    return _SC_STAGE_CACHE["k"]


def _tc_body(seg_ref, w_ref, b_ref, g_ref, beta_ref, tnp_ref, o_ref):
    seg = seg_ref[...].reshape(B, PROWS, T)
    wext = jnp.concatenate(
        [w_ref[...], b_ref[...][None, :],
         jnp.zeros((PROWS - PATCH - 1, D_MODEL), jnp.float32)], axis=0)
    pe = lax.dot_general(
        seg, wext, (((1,), (0,)), ((), ())),
        precision=lax.Precision.HIGHEST,
        preferred_element_type=jnp.float32)
    tok3 = lax.broadcasted_iota(jnp.int32, (B, T, D_MODEL), 1)
    pe = jnp.where(tok3 < tnp_ref[0], pe, 0.0)
    mean = jnp.mean(pe, axis=-1, keepdims=True)
    cen = pe - mean
    var = jnp.mean(cen * cen, axis=-1, keepdims=True)
    o_ref[...] = cen * lax.rsqrt(var + EPS) * g_ref[...] + beta_ref[...]


def kernel(x, target_n_patches, W, b, gamma, beta):
    seg = _sc_stage_fn()(x)
    tnp = jnp.asarray(target_n_patches, jnp.int32).reshape(1)
    out = pl.pallas_call(
        _tc_body,
        out_shape=jax.ShapeDtypeStruct((B, T, D_MODEL), jnp.float32),
        in_specs=[
            pl.BlockSpec(memory_space=pltpu.VMEM),
            pl.BlockSpec(memory_space=pltpu.VMEM),
            pl.BlockSpec(memory_space=pltpu.VMEM),
            pl.BlockSpec(memory_space=pltpu.VMEM),
            pl.BlockSpec(memory_space=pltpu.VMEM),
            pl.BlockSpec(memory_space=pltpu.SMEM),
        ],
    )(seg, W, b, gamma, beta, tnp)
    return out
